# initial kernel scaffold (unmeasured)
import jax
import jax.numpy as jnp
from jax import lax
from jax.experimental import pallas as pl
from jax.experimental.pallas import tpu as pltpu

N_DEV = 16
M_PER = 256
N_PER = 512
K = 4096


def kernel(x, w_mat):
    def body(x_ref, w_ref, out_ref,
             xb, wv, ybuf, comm,
             copy_sem, send_sems, recv_sems):
        my_i = lax.axis_index("i")

        xb[...] = x_ref[...].astype(jnp.bfloat16)

        for s in range(N_DEV):
            j = (my_i + s) % N_DEV
            cp = pltpu.make_async_copy(
                w_ref.at[:, pl.ds(j * N_PER, N_PER)],
                wv,
                copy_sem,
            )
            cp.start()
            cp.wait()
            y = jnp.dot(
                xb[...],
                wv[...].astype(jnp.bfloat16),
                preferred_element_type=jnp.float32,
            )
            y = jnp.maximum(y, 0.0)
            if s == 0:
                out_ref[pl.ds(my_i * M_PER, M_PER), :] = y
            else:
                ybuf[s, :, :] = y.astype(jnp.bfloat16)
                rdma = pltpu.make_async_remote_copy(
                    src_ref=ybuf.at[s],
                    dst_ref=comm.at[s],
                    send_sem=send_sems.at[s],
                    recv_sem=recv_sems.at[s],
                    device_id=(j,),
                    device_id_type=pl.DeviceIdType.MESH,
                )
                rdma.start()

        for s in range(1, N_DEV):
            src = (my_i + N_DEV - s) % N_DEV
            recv = pltpu.make_async_remote_copy(
                src_ref=ybuf.at[s],
                dst_ref=comm.at[s],
                send_sem=send_sems.at[s],
                recv_sem=recv_sems.at[s],
                device_id=(src,),
                device_id_type=pl.DeviceIdType.MESH,
            )
            recv.wait_recv()
            out_ref[pl.ds(src * M_PER, M_PER), :] = comm[s].astype(jnp.float32)

        for s in range(1, N_DEV):
            snd = pltpu.make_async_remote_copy(
                src_ref=ybuf.at[s],
                dst_ref=comm.at[s],
                send_sem=send_sems.at[s],
                recv_sem=recv_sems.at[s],
                device_id=((my_i + s) % N_DEV,),
                device_id_type=pl.DeviceIdType.MESH,
            )
            snd.wait_send()

    return pl.pallas_call(
        body,
        out_shape=jax.ShapeDtypeStruct((N_DEV * M_PER, N_PER), jnp.float32),
        in_specs=[
            pl.BlockSpec(memory_space=pltpu.VMEM),
            pl.BlockSpec(memory_space=pltpu.ANY),
        ],
        out_specs=pl.BlockSpec(memory_space=pltpu.VMEM),
        scratch_shapes=[
            pltpu.VMEM((M_PER, K), jnp.bfloat16),
            pltpu.VMEM((K, N_PER), jnp.float32),
            pltpu.VMEM((N_DEV, M_PER, N_PER), jnp.bfloat16),
            pltpu.VMEM((N_DEV, M_PER, N_PER), jnp.bfloat16),
            pltpu.SemaphoreType.DMA,
            pltpu.SemaphoreType.DMA((N_DEV,)),
            pltpu.SemaphoreType.DMA((N_DEV,)),
        ],
    )(x, w_mat)


# baseline (device time: 96071 ns/iter reference)
import jax
import jax.numpy as jnp
from jax import lax
from jax.experimental import pallas as pl
from jax.experimental.pallas import tpu as pltpu

N_DEV = 16
M_PER = 256
N_PER = 512
K = 4096


def kernel(x, w_mat):
    def body(x_ref, w_ref, out_ref,
             xb, wv, ybuf, comm,
             copy_sem, send_sems, recv_sems):
        my_i = lax.axis_index("i")

        xb[...] = x_ref[...].astype(jnp.bfloat16)

        for s in range(N_DEV):
            j = (my_i + s) % N_DEV
            cp = pltpu.make_async_copy(
                w_ref.at[:, pl.ds(j * N_PER, N_PER)],
                wv,
                copy_sem,
            )
            cp.start()
            cp.wait()
            y = jnp.dot(
                xb[...],
                wv[...].astype(jnp.bfloat16),
                preferred_element_type=jnp.float32,
            )
            y = jnp.maximum(y, 0.0)
            if s == 0:
                out_ref[pl.ds(my_i * M_PER, M_PER), :] = y
            else:
                ybuf[s, :, :] = y.astype(jnp.bfloat16)
                rdma = pltpu.make_async_remote_copy(
                    src_ref=ybuf.at[s],
                    dst_ref=comm.at[s],
                    send_sem=send_sems.at[s],
                    recv_sem=recv_sems.at[s],
                    device_id=(j,),
                    device_id_type=pl.DeviceIdType.MESH,
                )
                rdma.start()

        for s in range(1, N_DEV):
            src = (my_i + N_DEV - s) % N_DEV
            recv = pltpu.make_async_remote_copy(
                src_ref=ybuf.at[s],
                dst_ref=comm.at[s],
                send_sem=send_sems.at[s],
                recv_sem=recv_sems.at[s],
                device_id=(src,),
                device_id_type=pl.DeviceIdType.MESH,
            )
            recv.wait_recv()
            out_ref[pl.ds(src * M_PER, M_PER), :] = comm[s].astype(jnp.float32)

        for s in range(1, N_DEV):
            snd = pltpu.make_async_remote_copy(
                src_ref=ybuf.at[s],
                dst_ref=comm.at[s],
                send_sem=send_sems.at[s],
                recv_sem=recv_sems.at[s],
                device_id=((my_i + s) % N_DEV,),
                device_id_type=pl.DeviceIdType.MESH,
            )
            snd.wait_send()

    return pl.pallas_call(
        body,
        out_shape=jax.ShapeDtypeStruct((N_DEV * M_PER, N_PER), jnp.float32),
        in_specs=[
            pl.BlockSpec(memory_space=pltpu.VMEM),
            pl.BlockSpec(memory_space=pl.ANY),
        ],
        out_specs=pl.BlockSpec(memory_space=pltpu.VMEM),
        scratch_shapes=[
            pltpu.VMEM((M_PER, K), jnp.bfloat16),
            pltpu.VMEM((K, N_PER), jnp.float32),
            pltpu.VMEM((N_DEV, M_PER, N_PER), jnp.bfloat16),
            pltpu.VMEM((N_DEV, M_PER, N_PER), jnp.bfloat16),
            pltpu.SemaphoreType.DMA,
            pltpu.SemaphoreType.DMA((N_DEV,)),
            pltpu.SemaphoreType.DMA((N_DEV,)),
        ],
    )(x, w_mat)
